# Initial kernel scaffold; baseline (speedup 1.0000x reference)
import jax
import jax.numpy as jnp
from jax import lax
from jax.experimental import pallas as pl
from jax.experimental.pallas import tpu as pltpu
from jax.experimental.pallas import tpu_sc as plsc

N = 10000
E = 320000
D = 128

NC = 2
NS = 16
NW = NC * NS
EPW = E // NW
CH = 128
NCH = EPW // CH
TAIL = EPW - NCH * CH
RPT = N // NS
DG = D // 16


def _bcast_lane(v, i):
    idx = jnp.full((16,), i, jnp.int32)
    return jax.lax.gather(
        v, idx[:, None],
        dimension_numbers=jax.lax.GatherDimensionNumbers(
            offset_dims=(), collapsed_slice_dims=(0,), start_index_map=(0,)),
        slice_sizes=(1,),
        mode=jax.lax.GatherScatterMode.PROMISE_IN_BOUNDS)


def _scale_rows(gbuf, vals_v, voff, nrows16):
    def group(g, _):
        v16 = vals_v[pl.ds(voff + g * 16, 16)]
        for i in range(16):
            b = _bcast_lane(v16, i)
            r = g * 16 + i
            for k in range(DG):
                gbuf[r, pl.ds(k * 16, 16)] = gbuf[r, pl.ds(k * 16, 16)] * b
        return 0
    lax.fori_loop(0, nrows16, group, 0)


def _spmm_body(x_hbm, vals_hbm, rows_hbm, cols_hbm, part_hbm,
               acc, cols_v, rows_v, vals_v, gbuf):
    c = lax.axis_index("c")
    s = lax.axis_index("s")
    wid = s * NC + c
    base = wid * EPW

    def zrow(i, _):
        for k in range(DG):
            gbuf[i, pl.ds(k * 16, 16)] = jnp.zeros((16,), jnp.float32)
        return 0
    lax.fori_loop(0, CH, zrow, 0)
    for q in range(5):
        pltpu.sync_copy(gbuf.at[pl.ds(0, 125)],
                        acc.at[pl.ds(s * RPT + q * 125, 125)])

    pltpu.sync_copy(cols_hbm.at[pl.ds(base, EPW)], cols_v)
    pltpu.sync_copy(rows_hbm.at[pl.ds(base, EPW)], rows_v)
    pltpu.sync_copy(vals_hbm.at[pl.ds(base, EPW)], vals_v)

    plsc.subcore_barrier()

    def chunk(ci, _):
        off = ci * CH
        pltpu.sync_copy(x_hbm.at[cols_v.at[pl.ds(off, CH)]], gbuf)
        _scale_rows(gbuf, vals_v, off, CH // 16)

        def scatter(g, _):
            rvec = rows_v[pl.ds(off + g * 16, 16)]
            pltpu.sync_copy(gbuf.at[pl.ds(g * 16, 16)], acc.at[rvec],
                            add=True)
            return 0
        lax.fori_loop(0, CH // 16, scatter, 0)
        return 0
    lax.fori_loop(0, NCH, chunk, 0)

    toff = NCH * CH
    ctail = cols_v[pl.ds(toff, TAIL)]
    pltpu.sync_copy(x_hbm.at[ctail], gbuf.at[pl.ds(0, TAIL)])
    _scale_rows(gbuf, vals_v, toff, TAIL // 16)
    rtail = rows_v[pl.ds(toff, TAIL)]
    pltpu.sync_copy(gbuf.at[pl.ds(0, TAIL)], acc.at[rtail], add=True)

    plsc.subcore_barrier()
    pltpu.sync_copy(acc.at[pl.ds(s * RPT, RPT)],
                    part_hbm.at[c, pl.ds(s * RPT, RPT)])


_spmm_sc = pl.kernel(
    _spmm_body,
    out_type=jax.ShapeDtypeStruct((NC, N, D), jnp.float32),
    mesh=plsc.VectorSubcoreMesh(core_axis_name="c", subcore_axis_name="s",
                                num_cores=NC, num_subcores=NS),
    scratch_types=[
        pltpu.VMEM_SHARED((N, D), jnp.float32),
        pltpu.VMEM((EPW,), jnp.int32),
        pltpu.VMEM((EPW,), jnp.int32),
        pltpu.VMEM((EPW,), jnp.float32),
        pltpu.VMEM((CH, D), jnp.float32),
    ],
)


def _sum2_body(p_ref, o_ref):
    o_ref[...] = p_ref[0] + p_ref[1]


_BLK = 400

_sum2 = pl.pallas_call(
    _sum2_body,
    grid=(N // _BLK,),
    in_specs=[pl.BlockSpec((NC, _BLK, D), lambda i: (0, i, 0))],
    out_specs=pl.BlockSpec((_BLK, D), lambda i: (i, 0)),
    out_shape=jax.ShapeDtypeStruct((N, D), jnp.float32),
)


def kernel(x, edge_vals, edge_index):
    rows = edge_index[0]
    cols = edge_index[1]
    part = _spmm_sc(x, edge_vals, rows, cols)
    return _sum2(part)


# speedup vs baseline: 6.7518x; 6.7518x over previous
import jax
import jax.numpy as jnp
from jax import lax
from jax.experimental import pallas as pl
from jax.experimental.pallas import tpu as pltpu
from jax.experimental.pallas import tpu_sc as plsc

N = 10000
E = 320000
D = 128

NC = 2
NS = 16
NW = NC * NS
EPW = E // NW
CH = 128
NCH = EPW // CH
TAIL = EPW - NCH * CH
RPT = 624
DG = D // 16


def _bcast_lane(v, i):
    idx = jnp.full((16,), i, jnp.int32)
    return jax.lax.gather(
        v, idx[:, None],
        dimension_numbers=jax.lax.GatherDimensionNumbers(
            offset_dims=(), collapsed_slice_dims=(0,), start_index_map=(0,)),
        slice_sizes=(1,),
        mode=jax.lax.GatherScatterMode.PROMISE_IN_BOUNDS)


def _scale_rows(gbuf, vals_v, voff, nrows16):
    def group(g, _):
        v16 = vals_v[pl.ds(voff + g * 16, 16)]
        for i in range(16):
            b = _bcast_lane(v16, i)
            r = g * 16 + i
            for k in range(DG):
                gbuf[r, pl.ds(k * 16, 16)] = gbuf[r, pl.ds(k * 16, 16)] * b
        return 0
    lax.fori_loop(0, nrows16, group, 0)


def _spmm_body(x_hbm, vals_hbm, rows_hbm, cols_hbm, part_hbm,
               acc, cols_v, rows_v, vals_v, gbuf):
    c = lax.axis_index("c")
    s = lax.axis_index("s")
    wid = s * NC + c
    base = wid * EPW

    def zrow(i, _):
        for k in range(DG):
            gbuf[i, pl.ds(k * 16, 16)] = jnp.zeros((16,), jnp.float32)
        return 0
    lax.fori_loop(0, CH, zrow, 0)
    for q in range(4):
        pltpu.sync_copy(gbuf.at[pl.ds(0, CH)],
                        acc.at[pl.ds(s * RPT + q * CH, CH)])
    pltpu.sync_copy(gbuf.at[pl.ds(0, 112)],
                    acc.at[pl.ds(s * RPT + 4 * CH, 112)])

    @pl.when(s == NS - 1)
    def _zero_last():
        pltpu.sync_copy(gbuf.at[pl.ds(0, 16)],
                        acc.at[pl.ds(NS * RPT, 16)])

    pltpu.sync_copy(cols_hbm.at[pl.ds(base, EPW)], cols_v)
    pltpu.sync_copy(rows_hbm.at[pl.ds(base, EPW)], rows_v)
    pltpu.sync_copy(vals_hbm.at[pl.ds(base, EPW)], vals_v)

    plsc.subcore_barrier()

    def chunk(ci, _):
        off = ci * CH
        pltpu.sync_copy(x_hbm.at[cols_v.at[pl.ds(off, CH)]], gbuf)
        _scale_rows(gbuf, vals_v, off, CH // 16)

        def scatter(g, _):
            rvec = rows_v[pl.ds(off + g * 16, 16)]
            pltpu.sync_copy(gbuf.at[pl.ds(g * 16, 16)], acc.at[rvec],
                            add=True)
            return 0
        lax.fori_loop(0, CH // 16, scatter, 0)
        return 0
    lax.fori_loop(0, NCH, chunk, 0)

    toff = NCH * CH
    ctail = cols_v[pl.ds(toff, TAIL)]
    pltpu.sync_copy(x_hbm.at[ctail], gbuf.at[pl.ds(0, TAIL)])
    _scale_rows(gbuf, vals_v, toff, TAIL // 16)
    rtail = rows_v[pl.ds(toff, TAIL)]
    pltpu.sync_copy(gbuf.at[pl.ds(0, TAIL)], acc.at[rtail], add=True)

    plsc.subcore_barrier()
    pltpu.sync_copy(acc.at[pl.ds(s * RPT, RPT)],
                    part_hbm.at[c, pl.ds(s * RPT, RPT)])

    @pl.when(s == NS - 1)
    def _dump_last():
        pltpu.sync_copy(acc.at[pl.ds(NS * RPT, 16)],
                        part_hbm.at[c, pl.ds(NS * RPT, 16)])


_spmm_sc = pl.kernel(
    _spmm_body,
    out_type=jax.ShapeDtypeStruct((NC, N, D), jnp.float32),
    mesh=plsc.VectorSubcoreMesh(core_axis_name="c", subcore_axis_name="s",
                                num_cores=NC, num_subcores=NS),
    scratch_types=[
        pltpu.VMEM_SHARED((N, D), jnp.float32),
        pltpu.VMEM((EPW,), jnp.int32),
        pltpu.VMEM((EPW,), jnp.int32),
        pltpu.VMEM((EPW,), jnp.float32),
        pltpu.VMEM((CH, D), jnp.float32),
    ],
)


def _sum2_body(p_ref, o_ref):
    o_ref[...] = p_ref[0] + p_ref[1]


_BLK = 400

_sum2 = pl.pallas_call(
    _sum2_body,
    grid=(N // _BLK,),
    in_specs=[pl.BlockSpec((NC, _BLK, D), lambda i: (0, i, 0))],
    out_specs=pl.BlockSpec((_BLK, D), lambda i: (i, 0)),
    out_shape=jax.ShapeDtypeStruct((N, D), jnp.float32),
)


def kernel(x, edge_vals, edge_index):
    rows = edge_index[0]
    cols = edge_index[1]
    part = _spmm_sc(x, edge_vals, rows, cols)
    return _sum2(part)


# speedup vs baseline: 9.1860x; 1.3605x over previous
import jax
import jax.numpy as jnp
from jax import lax
from jax.experimental import pallas as pl
from jax.experimental.pallas import tpu as pltpu
from jax.experimental.pallas import tpu_sc as plsc

N = 10000
E = 320000
D = 128

NC = 2
NS = 16
NW = NC * NS
EPW = E // NW
CH = 64
NCH = EPW // CH
TAIL = EPW - NCH * CH
RPT = 624
DG = D // 16


def _bcast_lane(v, i):
    idx = jnp.full((16,), i, jnp.int32)
    return jax.lax.gather(
        v, idx[:, None],
        dimension_numbers=jax.lax.GatherDimensionNumbers(
            offset_dims=(), collapsed_slice_dims=(0,), start_index_map=(0,)),
        slice_sizes=(1,),
        mode=jax.lax.GatherScatterMode.PROMISE_IN_BOUNDS)


def _scale_group(gbuf, vals_v, voff, g):
    v16 = vals_v[pl.ds(voff + g * 16, 16)]
    for i in range(16):
        b = _bcast_lane(v16, i)
        r = g * 16 + i
        for k in range(DG):
            gbuf[r, pl.ds(k * 16, 16)] = gbuf[r, pl.ds(k * 16, 16)] * b


def _scale_rows(gbuf, vals_v, voff, nrows16):
    def group(g, _):
        _scale_group(gbuf, vals_v, voff, g)
        return 0
    lax.fori_loop(0, nrows16, group, 0)


def _spmm_body(x_hbm, vals_hbm, rows_hbm, cols_hbm, part_hbm,
               acc, cols_v, rows_v, vals_v, gbuf, gsem, ssem):
    c = lax.axis_index("c")
    s = lax.axis_index("s")
    wid = s * NC + c
    base = wid * EPW

    def zrow(i, _):
        for k in range(DG):
            gbuf[0, i, pl.ds(k * 16, 16)] = jnp.zeros((16,), jnp.float32)
        return 0
    lax.fori_loop(0, CH, zrow, 0)
    for q in range(RPT // CH):
        pltpu.sync_copy(gbuf.at[0],
                        acc.at[pl.ds(s * RPT + q * CH, CH)])
    _zrem = RPT - (RPT // CH) * CH
    if _zrem:
        pltpu.sync_copy(gbuf.at[0, pl.ds(0, _zrem)],
                        acc.at[pl.ds(s * RPT + (RPT // CH) * CH, _zrem)])

    @pl.when(s == NS - 1)
    def _zero_last():
        pltpu.sync_copy(gbuf.at[0, pl.ds(0, 16)],
                        acc.at[pl.ds(NS * RPT, 16)])

    pltpu.sync_copy(cols_hbm.at[pl.ds(base, EPW)], cols_v)
    pltpu.sync_copy(rows_hbm.at[pl.ds(base, EPW)], rows_v)
    pltpu.sync_copy(vals_hbm.at[pl.ds(base, EPW)], vals_v)

    pltpu.async_copy(x_hbm.at[cols_v.at[pl.ds(0, CH)]], gbuf.at[0], gsem.at[0])

    plsc.subcore_barrier()

    def half(ci, p):
        gb = gbuf.at[p]
        off = ci * CH
        pltpu.make_async_copy(
            x_hbm.at[cols_v.at[pl.ds(off, CH)]], gb, gsem.at[p]).wait()
        @pl.when(ci + 1 < NCH)
        def _prefetch():
            noff = (ci + 1) * CH
            pltpu.async_copy(x_hbm.at[cols_v.at[pl.ds(noff, CH)]],
                             gbuf.at[1 - p], gsem.at[1 - p])
        scat = []
        for g in range(CH // 16):
            _scale_group(gb, vals_v, off, g)
            rvec = rows_v[pl.ds(off + g * 16, 16)]
            scat.append(pltpu.async_copy(
                gb.at[pl.ds(g * 16, 16)], acc.at[rvec], ssem.at[p], add=True))
        for d in scat:
            d.wait()

    def pair(j, _):
        half(j * 2, 0)
        half(j * 2 + 1, 1)
        return 0
    lax.fori_loop(0, NCH // 2, pair, 0)

    toff = NCH * CH
    ctail = cols_v[pl.ds(toff, TAIL)]
    pltpu.sync_copy(x_hbm.at[ctail], gbuf.at[0, pl.ds(0, TAIL)])
    _scale_rows(gbuf.at[0], vals_v, toff, TAIL // 16)
    rtail = rows_v[pl.ds(toff, TAIL)]
    pltpu.sync_copy(gbuf.at[0, pl.ds(0, TAIL)], acc.at[rtail], add=True)

    plsc.subcore_barrier()
    pltpu.sync_copy(acc.at[pl.ds(s * RPT, RPT)],
                    part_hbm.at[c, pl.ds(s * RPT, RPT)])

    @pl.when(s == NS - 1)
    def _dump_last():
        pltpu.sync_copy(acc.at[pl.ds(NS * RPT, 16)],
                        part_hbm.at[c, pl.ds(NS * RPT, 16)])


_spmm_sc = pl.kernel(
    _spmm_body,
    out_type=jax.ShapeDtypeStruct((NC, N, D), jnp.float32),
    mesh=plsc.VectorSubcoreMesh(core_axis_name="c", subcore_axis_name="s",
                                num_cores=NC, num_subcores=NS),
    scratch_types=[
        pltpu.VMEM_SHARED((N, D), jnp.float32),
        pltpu.VMEM((EPW,), jnp.int32),
        pltpu.VMEM((EPW,), jnp.int32),
        pltpu.VMEM((EPW,), jnp.float32),
        pltpu.VMEM((2, CH, D), jnp.float32),
        pltpu.SemaphoreType.DMA((2,)),
        pltpu.SemaphoreType.DMA((2,)),
    ],
)


def _sum2_body(p_ref, o_ref):
    o_ref[...] = p_ref[0] + p_ref[1]


_BLK = 400

_sum2 = pl.pallas_call(
    _sum2_body,
    grid=(N // _BLK,),
    in_specs=[pl.BlockSpec((NC, _BLK, D), lambda i: (0, i, 0))],
    out_specs=pl.BlockSpec((_BLK, D), lambda i: (i, 0)),
    out_shape=jax.ShapeDtypeStruct((N, D), jnp.float32),
)


def kernel(x, edge_vals, edge_index):
    rows = edge_index[0]
    cols = edge_index[1]
    part = _spmm_sc(x, edge_vals, rows, cols)
    return _sum2(part)


# speedup vs baseline: 9.5786x; 1.0427x over previous
import jax
import jax.numpy as jnp
from jax import lax
from jax.experimental import pallas as pl
from jax.experimental.pallas import tpu as pltpu
from jax.experimental.pallas import tpu_sc as plsc

N = 10000
E = 320000
D = 128

NC = 2
NS = 16
NW = NC * NS
EPW = E // NW
CH = 128
NR = 4
NCH = EPW // CH
TAIL = EPW - NCH * CH
RPT = 624
DG = D // 16


def _bcast_lane(v, i):
    idx = jnp.full((16,), i, jnp.int32)
    return jax.lax.gather(
        v, idx[:, None],
        dimension_numbers=jax.lax.GatherDimensionNumbers(
            offset_dims=(), collapsed_slice_dims=(0,), start_index_map=(0,)),
        slice_sizes=(1,),
        mode=jax.lax.GatherScatterMode.PROMISE_IN_BOUNDS)


def _scale_group(gb, vring, q, g):
    v16 = vring[q, pl.ds(g * 16, 16)]
    for i in range(16):
        b = _bcast_lane(v16, i)
        r = g * 16 + i
        for k in range(DG):
            gb[r, pl.ds(k * 16, 16)] = gb[r, pl.ds(k * 16, 16)] * b


def _spmm_body(x_hbm, vals_hbm, rows_hbm, cols_hbm, part_hbm,
               acc, cring, rring, vring, gbuf, gsem, ssem, esem):
    c = lax.axis_index("c")
    s = lax.axis_index("s")
    wid = s * NC + c
    base = wid * EPW

    def estage(ci):
        q = lax.rem(ci, NR)
        off = base + ci * CH
        pltpu.async_copy(cols_hbm.at[pl.ds(off, CH)], cring.at[q], esem.at[q])
        pltpu.async_copy(rows_hbm.at[pl.ds(off, CH)], rring.at[q], esem.at[q])
        pltpu.async_copy(vals_hbm.at[pl.ds(off, CH)], vring.at[q], esem.at[q])

    def estage_wait(ci):
        q = lax.rem(ci, NR)
        off = base + ci * CH
        pltpu.make_async_copy(cols_hbm.at[pl.ds(off, CH)], cring.at[q],
                              esem.at[q]).wait()
        pltpu.make_async_copy(rows_hbm.at[pl.ds(off, CH)], rring.at[q],
                              esem.at[q]).wait()
        pltpu.make_async_copy(vals_hbm.at[pl.ds(off, CH)], vring.at[q],
                              esem.at[q]).wait()

    def zrow(i, _):
        for k in range(DG):
            gbuf[0, i, pl.ds(k * 16, 16)] = jnp.zeros((16,), jnp.float32)
        return 0
    lax.fori_loop(0, CH, zrow, 0)
    for q in range(RPT // CH):
        pltpu.sync_copy(gbuf.at[0],
                        acc.at[pl.ds(s * RPT + q * CH, CH)])
    _zrem = RPT - (RPT // CH) * CH
    if _zrem:
        pltpu.sync_copy(gbuf.at[0, pl.ds(0, _zrem)],
                        acc.at[pl.ds(s * RPT + (RPT // CH) * CH, _zrem)])

    @pl.when(s == NS - 1)
    def _zero_last():
        pltpu.sync_copy(gbuf.at[0, pl.ds(0, 16)],
                        acc.at[pl.ds(NS * RPT, 16)])

    estage(0)
    estage(1)
    estage_wait(0)
    pltpu.async_copy(x_hbm.at[cring.at[0]], gbuf.at[0], gsem.at[0])

    plsc.subcore_barrier()

    _iota16 = jax.lax.broadcasted_iota(jnp.int32, (16,), 0)

    def half(ci, p):
        gb = gbuf.at[p]
        q = lax.rem(ci, NR)
        @pl.when(ci >= 1)
        def _drain():
            for g in range(CH // 16):
                pltpu.make_async_copy(
                    gbuf.at[1 - p].at[pl.ds(g * 16, 16)],
                    acc.at[_iota16], ssem.at[1 - p]).wait()
        @pl.when(ci + 1 < NCH)
        def _prefetch():
            qn = lax.rem(ci + 1, NR)
            estage_wait(ci + 1)
            pltpu.async_copy(x_hbm.at[cring.at[qn]], gbuf.at[1 - p],
                             gsem.at[1 - p])
        @pl.when(ci + 2 < NCH)
        def _stage():
            estage(ci + 2)
        pltpu.make_async_copy(x_hbm.at[cring.at[q]], gb, gsem.at[p]).wait()
        for g in range(CH // 16):
            _scale_group(gb, vring, q, g)
            rvec = rring[q, pl.ds(g * 16, 16)]
            pltpu.async_copy(gb.at[pl.ds(g * 16, 16)], acc.at[rvec],
                             ssem.at[p], add=True)

    def pair(j, _):
        half(j * 2, 0)
        half(j * 2 + 1, 1)
        return 0
    lax.fori_loop(0, NCH // 2, pair, 0)

    for g in range(CH // 16):
        pltpu.make_async_copy(gbuf.at[1].at[pl.ds(g * 16, 16)],
                              acc.at[_iota16], ssem.at[1]).wait()

    toff = base + NCH * CH
    pltpu.sync_copy(cols_hbm.at[pl.ds(toff, TAIL)], cring.at[0, pl.ds(0, TAIL)])
    pltpu.sync_copy(rows_hbm.at[pl.ds(toff, TAIL)], rring.at[0, pl.ds(0, TAIL)])
    pltpu.sync_copy(vals_hbm.at[pl.ds(toff, TAIL)], vring.at[0, pl.ds(0, TAIL)])
    ctail = cring[0, pl.ds(0, TAIL)]
    pltpu.sync_copy(x_hbm.at[ctail], gbuf.at[0, pl.ds(0, TAIL)])
    _scale_group(gbuf.at[0], vring, 0, 0)
    rtail = rring[0, pl.ds(0, TAIL)]
    pltpu.sync_copy(gbuf.at[0, pl.ds(0, TAIL)], acc.at[rtail], add=True)

    plsc.subcore_barrier()
    pltpu.sync_copy(acc.at[pl.ds(s * RPT, RPT)],
                    part_hbm.at[c, pl.ds(s * RPT, RPT)])

    @pl.when(s == NS - 1)
    def _dump_last():
        pltpu.sync_copy(acc.at[pl.ds(NS * RPT, 16)],
                        part_hbm.at[c, pl.ds(NS * RPT, 16)])


_spmm_sc = pl.kernel(
    _spmm_body,
    out_type=jax.ShapeDtypeStruct((NC, N, D), jnp.float32),
    mesh=plsc.VectorSubcoreMesh(core_axis_name="c", subcore_axis_name="s",
                                num_cores=NC, num_subcores=NS),
    scratch_types=[
        pltpu.VMEM_SHARED((N, D), jnp.float32),
        pltpu.VMEM((NR, CH), jnp.int32),
        pltpu.VMEM((NR, CH), jnp.int32),
        pltpu.VMEM((NR, CH), jnp.float32),
        pltpu.VMEM((2, CH, D), jnp.float32),
        pltpu.SemaphoreType.DMA((2,)),
        pltpu.SemaphoreType.DMA((2,)),
        pltpu.SemaphoreType.DMA((NR,)),
    ],
)


def _sum2_body(p_ref, o_ref):
    o_ref[...] = p_ref[0] + p_ref[1]


_BLK = 400

_sum2 = pl.pallas_call(
    _sum2_body,
    grid=(N // _BLK,),
    in_specs=[pl.BlockSpec((NC, _BLK, D), lambda i: (0, i, 0))],
    out_specs=pl.BlockSpec((_BLK, D), lambda i: (i, 0)),
    out_shape=jax.ShapeDtypeStruct((N, D), jnp.float32),
)


def kernel(x, edge_vals, edge_index):
    rows = edge_index[0]
    cols = edge_index[1]
    part = _spmm_sc(x, edge_vals, rows, cols)
    return _sum2(part)
